# baseline (speedup 1.0000x reference)
import functools

import jax
import jax.numpy as jnp
import numpy as np
from jax.experimental import pallas as pl
from jax.experimental.pallas import tpu as pltpu

_HUFFMAN_PATHS = ((0, 1), (1, 0), (0, 0, 1), (1, 1, 0))
_HIDDEN = 512
_NCOL = 16


def _tables(batch: int):
    m = np.zeros((8, _NCOL), np.float32)
    t = np.zeros((8, _NCOL), np.float32)
    off = 0
    for w, path in enumerate(_HUFFMAN_PATHS):
        for j, bit in enumerate(path):
            m[w, off + j] = 1.0 / (len(path) * batch)
            t[w, off + j] = float(bit)
        off += len(path)
    return m, t


def _body(h_ref, tw_ref, w_ref, m_ref, t_ref, out_ref):
    bm = h_ref.shape[0]
    x = jnp.dot(h_ref[...], w_ref[...], preferred_element_type=jnp.float32)
    tw = tw_ref[...]
    onehot = (tw == jax.lax.broadcasted_iota(jnp.int32, (bm, 8), 1)).astype(
        jnp.float32
    )
    mask = jnp.dot(onehot, m_ref[...], preferred_element_type=jnp.float32)
    tgt = jnp.dot(onehot, t_ref[...], preferred_element_type=jnp.float32)
    loss = jnp.maximum(x, 0.0) - x * tgt + jnp.log1p(jnp.exp(-jnp.abs(x)))
    part = jnp.sum(mask * loss)

    @pl.when(pl.program_id(0) == 0)
    def _():
        out_ref[0, 0] = 0.0

    out_ref[0, 0] += part


@functools.partial(jax.jit, static_argnames=("interpret",))
def kernel(hidden, target_words, W_0, W_1, W_2, W_3, interpret=False):
    batch, hdim = hidden.shape
    bm = 512
    grid = batch // bm

    wstack = jnp.concatenate([W_0, W_1, W_2, W_3], axis=0)
    wstack = jnp.pad(wstack, ((0, _NCOL - wstack.shape[0]), (0, 0)))
    wt = wstack.T

    m_np, t_np = _tables(batch)
    m_tab = jnp.asarray(m_np)
    t_tab = jnp.asarray(t_np)
    tw2d = target_words.astype(jnp.int32).reshape(batch, 1)

    out = pl.pallas_call(
        _body,
        grid=(grid,),
        in_specs=[
            pl.BlockSpec((bm, hdim), lambda i: (i, 0)),
            pl.BlockSpec((bm, 1), lambda i: (i, 0)),
            pl.BlockSpec((hdim, _NCOL), lambda i: (0, 0)),
            pl.BlockSpec((8, _NCOL), lambda i: (0, 0)),
            pl.BlockSpec((8, _NCOL), lambda i: (0, 0)),
        ],
        out_specs=pl.BlockSpec(
            (1, 1), lambda i: (0, 0), memory_space=pltpu.SMEM
        ),
        out_shape=jax.ShapeDtypeStruct((1, 1), jnp.float32),
        interpret=interpret,
    )(hidden, tw2d, wt, m_tab, t_tab)
    return out[0, 0]
